# baseline (device time: 8070 ns/iter reference)
import jax
import jax.numpy as jnp
from jax import lax
from jax.experimental import pallas as pl
from jax.experimental.pallas import tpu as pltpu

X_SIZE = 2
ROW_CHUNK = 256
N_HALF = 2


def kernel(x):
    m_per, n_per = x.shape
    m_global = X_SIZE * m_per
    n_chunks = m_per // ROW_CHUNK
    n_half = n_per // N_HALF

    def body(x_ref, out_ref, comm_ref, send_sems, recv_sems):
        my_x = lax.axis_index("x")
        my_y = lax.axis_index("y")
        peer = (1 - my_x, my_y)

        barrier_sem = pltpu.get_barrier_semaphore()
        pl.semaphore_signal(
            barrier_sem, inc=1, device_id=peer,
            device_id_type=pl.DeviceIdType.MESH,
        )

        rdmas = []
        for h in range(N_HALF):
            cols = pl.ds(h * n_half, n_half)
            acc = jnp.zeros((1, n_half), dtype=x_ref.dtype)
            for k in range(n_chunks):
                acc += jnp.sum(
                    x_ref[pl.ds(k * ROW_CHUNK, ROW_CHUNK), cols],
                    axis=0, keepdims=True,
                )
            comm_ref[0, :, cols] = acc
            if h == 0:
                pl.semaphore_wait(barrier_sem, 1)
            rdma = pltpu.make_async_remote_copy(
                src_ref=comm_ref.at[0, :, cols],
                dst_ref=comm_ref.at[1, :, cols],
                send_sem=send_sems.at[h],
                recv_sem=recv_sems.at[h],
                device_id=peer,
                device_id_type=pl.DeviceIdType.MESH,
            )
            rdma.start()
            rdmas.append(rdma)

        for rdma in rdmas:
            rdma.wait()

        out_ref[:, :] = (comm_ref[0, :, :] + comm_ref[1, :, :]) * (
            1.0 / m_global
        )

    return pl.pallas_call(
        body,
        out_shape=jax.ShapeDtypeStruct((1, n_per), x.dtype),
        in_specs=[pl.BlockSpec(memory_space=pltpu.VMEM)],
        out_specs=pl.BlockSpec(memory_space=pltpu.VMEM),
        scratch_shapes=[
            pltpu.VMEM((2, 1, n_per), x.dtype),
            pltpu.SemaphoreType.DMA((N_HALF,)),
            pltpu.SemaphoreType.DMA((N_HALF,)),
        ],
        compiler_params=pltpu.CompilerParams(collective_id=0),
    )(x)
